# Initial kernel scaffold; baseline (speedup 1.0000x reference)
import functools

import jax
import jax.numpy as jnp
from jax import lax
from jax.experimental import pallas as pl
from jax.experimental.pallas import tpu as pltpu
from jax.experimental.pallas import tpu_sc as plsc

N = 50000
E = 1_600_000
NC = 2
NS = 16
NW = NC * NS
EPW = E // NW
B = 2000
G = B // 16
NCH = EPW // B
FW = 3 * B
VECS = FW // 16


def _body(r_hbm, off_hbm, ii_hbm, ij_hbm, out_hbm,
          rsh, ii_v, ij_v, ti_v, tj_v, gi_v, gj_v, off_v, sem):
    c = lax.axis_index("c")
    s = lax.axis_index("s")
    wid = s * NC + c

    @pl.when(s == 0)
    def _():
        pltpu.sync_copy(r_hbm, rsh)

    plsc.subcore_barrier()

    iota = lax.iota(jnp.int32, 16)
    ebase = wid * EPW

    def chunk(ch, carry):
        eb = ebase + ch * B
        pltpu.sync_copy(ii_hbm.at[pl.ds(eb, B)], ii_v)
        pltpu.sync_copy(ij_hbm.at[pl.ds(eb, B)], ij_v)
        pltpu.sync_copy(off_hbm.at[pl.ds(eb * 3, FW)], off_v)

        def expand(g, carry2):
            b16 = g * 16
            vi = ii_v[pl.ds(b16, 16)] * 3
            vj = ij_v[pl.ds(b16, 16)] * 3
            pos = iota * 3 + b16 * 3
            plsc.store_scatter(ti_v, [pos], vi)
            plsc.store_scatter(tj_v, [pos], vj)
            plsc.store_scatter(ti_v, [pos + 1], vi + 1)
            plsc.store_scatter(tj_v, [pos + 1], vj + 1)
            plsc.store_scatter(ti_v, [pos + 2], vi + 2)
            plsc.store_scatter(tj_v, [pos + 2], vj + 2)
            return carry2

        lax.fori_loop(0, G, expand, 0)

        pltpu.async_copy(rsh.at[tj_v], gj_v, sem).wait()
        pltpu.async_copy(rsh.at[ti_v], gi_v, sem).wait()

        def ew(v, carry2):
            sl = pl.ds(v * 16, 16)
            gj_v[sl] = gj_v[sl] - gi_v[sl] + off_v[sl]
            return carry2

        lax.fori_loop(0, VECS, ew, 0)

        pltpu.sync_copy(gj_v, out_hbm.at[pl.ds(eb * 3, FW)])
        return carry

    lax.fori_loop(0, NCH, chunk, 0)


@functools.partial(
    pl.kernel,
    mesh=plsc.VectorSubcoreMesh(core_axis_name="c", subcore_axis_name="s"),
    out_type=jax.ShapeDtypeStruct((E * 3,), jnp.float32),
    scratch_types=[
        pltpu.VMEM_SHARED((N * 3,), jnp.float32),
        pltpu.VMEM((B,), jnp.int32),
        pltpu.VMEM((B,), jnp.int32),
        pltpu.VMEM((FW,), jnp.int32),
        pltpu.VMEM((FW,), jnp.int32),
        pltpu.VMEM((FW,), jnp.float32),
        pltpu.VMEM((FW,), jnp.float32),
        pltpu.VMEM((FW,), jnp.float32),
        pltpu.SemaphoreType.DMA,
    ],
)
def _pairwise_sc(r_hbm, off_hbm, ii_hbm, ij_hbm, out_hbm, *scratch):
    _body(r_hbm, off_hbm, ii_hbm, ij_hbm, out_hbm, *scratch)


@jax.jit
def kernel(R, offsets, idx_i, idx_j):
    rf = R.reshape(-1)
    off_f = offsets.reshape(-1)
    ii = idx_i.astype(jnp.int32)
    ij = idx_j.astype(jnp.int32)
    out = _pairwise_sc(rf, off_f, ii, ij)
    return out.reshape(E, 3)


# speedup vs baseline: 1.3206x; 1.3206x over previous
import functools

import jax
import jax.numpy as jnp
from jax import lax
from jax.experimental import pallas as pl
from jax.experimental.pallas import tpu as pltpu
from jax.experimental.pallas import tpu_sc as plsc

N = 50000
E = 1_600_000
NC = 2
NS = 16
NW = NC * NS
EPW = E // NW
B = 2000
G = B // 16
NCH = EPW // B
FW = 3 * B
VECS = FW // 16


def _body(r_hbm, off_hbm, ii_hbm, ij_hbm, out_hbm,
          rsh, ii_v, ij_v, ti_v, tj_v, gi_v, gj_v, off_v, sem):
    c = lax.axis_index("c")
    s = lax.axis_index("s")
    wid = s * NC + c

    @pl.when(s == 0)
    def _():
        pltpu.sync_copy(r_hbm, rsh)

    plsc.subcore_barrier()

    iota = lax.iota(jnp.int32, 16)
    ebase = wid * EPW

    def chunk(ch, carry):
        eb = ebase + ch * B
        pltpu.sync_copy(ii_hbm.at[pl.ds(eb, B)], ii_v)
        pltpu.sync_copy(ij_hbm.at[pl.ds(eb, B)], ij_v)
        pltpu.sync_copy(off_hbm.at[pl.ds(eb * 3, FW)], off_v)

        def expand(g, carry2):
            b16 = g * 16
            vi = ii_v[pl.ds(b16, 16)] * 3
            vj = ij_v[pl.ds(b16, 16)] * 3
            pos = iota * 3 + b16 * 3
            plsc.store_scatter(ti_v, [pos], vi)
            plsc.store_scatter(tj_v, [pos], vj)
            plsc.store_scatter(ti_v, [pos + 1], vi + 1)
            plsc.store_scatter(tj_v, [pos + 1], vj + 1)
            plsc.store_scatter(ti_v, [pos + 2], vi + 2)
            plsc.store_scatter(tj_v, [pos + 2], vj + 2)
            return carry2

        lax.fori_loop(0, G, expand, 0)

        pltpu.async_copy(rsh.at[tj_v], gj_v, sem).wait()
        pltpu.async_copy(rsh.at[ti_v], gi_v, sem).wait()

        def ew(v, carry2):
            sl = pl.ds(v * 16, 16)
            gj_v[sl] = gj_v[sl] - gi_v[sl] + off_v[sl]
            return carry2

        lax.fori_loop(0, VECS, ew, 0)

        pltpu.sync_copy(gj_v, out_hbm.at[pl.ds(eb * 3, FW)])
        return carry

    lax.fori_loop(0, NCH, chunk, 0)


@functools.partial(
    pl.kernel,
    mesh=plsc.VectorSubcoreMesh(core_axis_name="c", subcore_axis_name="s"),
    out_type=jax.ShapeDtypeStruct((E * 3,), jnp.float32),
    compiler_params=pltpu.CompilerParams(needs_layout_passes=False),
    scratch_types=[
        pltpu.VMEM_SHARED((N * 3,), jnp.float32),
        pltpu.VMEM((B,), jnp.int32),
        pltpu.VMEM((B,), jnp.int32),
        pltpu.VMEM((FW,), jnp.int32),
        pltpu.VMEM((FW,), jnp.int32),
        pltpu.VMEM((FW,), jnp.float32),
        pltpu.VMEM((FW,), jnp.float32),
        pltpu.VMEM((FW,), jnp.float32),
        pltpu.SemaphoreType.DMA,
    ],
)
def _pairwise_sc(r_hbm, off_hbm, ii_hbm, ij_hbm, out_hbm, *scratch):
    _body(r_hbm, off_hbm, ii_hbm, ij_hbm, out_hbm, *scratch)


@jax.jit
def kernel(R, offsets, idx_i, idx_j):
    rf = R.reshape(-1)
    off_f = offsets.reshape(-1)
    ii = idx_i.astype(jnp.int32)
    ij = idx_j.astype(jnp.int32)
    out = _pairwise_sc(rf, off_f, ii, ij)
    return out.reshape(E, 3)


# speedup vs baseline: 21.6496x; 16.3941x over previous
import functools

import jax
import jax.numpy as jnp
from jax import lax
from jax.experimental import pallas as pl
from jax.experimental.pallas import tpu as pltpu
from jax.experimental.pallas import tpu_sc as plsc

N = 50000
E = 1_600_000
NC = 2
NS = 16
NW = NC * NS
EPW = E // NW
B = 2000
NCH = EPW // B
VECS = B // 16


def _body(rx_hbm, ry_hbm, rz_hbm, ox_hbm, oy_hbm, oz_hbm, ii_hbm, ij_hbm,
          outx_hbm, outy_hbm, outz_hbm,
          tx, ty, tz, ii_v, ij_v, gix, giy, giz, gjx, gjy, gjz,
          ofx, ofy, ofz, sem):
    c = lax.axis_index("c")
    s = lax.axis_index("s")
    wid = s * NC + c

    @pl.when(s == 0)
    def _():
        pltpu.sync_copy(rx_hbm, tx)

    @pl.when(s == 1)
    def _():
        pltpu.sync_copy(ry_hbm, ty)

    @pl.when(s == 2)
    def _():
        pltpu.sync_copy(rz_hbm, tz)

    plsc.subcore_barrier()

    ebase = wid * EPW

    def chunk(ch, carry):
        eb = ebase + ch * B
        sl = pl.ds(eb, B)
        pltpu.sync_copy(ii_hbm.at[sl], ii_v)
        pltpu.sync_copy(ij_hbm.at[sl], ij_v)
        pltpu.sync_copy(ox_hbm.at[sl], ofx)
        pltpu.sync_copy(oy_hbm.at[sl], ofy)
        pltpu.sync_copy(oz_hbm.at[sl], ofz)

        cj1 = pltpu.async_copy(tx.at[ij_v], gjx, sem)
        cj2 = pltpu.async_copy(ty.at[ij_v], gjy, sem)
        cj3 = pltpu.async_copy(tz.at[ij_v], gjz, sem)
        ci1 = pltpu.async_copy(tx.at[ii_v], gix, sem)
        ci2 = pltpu.async_copy(ty.at[ii_v], giy, sem)
        ci3 = pltpu.async_copy(tz.at[ii_v], giz, sem)
        cj1.wait()
        cj2.wait()
        cj3.wait()
        ci1.wait()
        ci2.wait()
        ci3.wait()

        def ew(v, carry2):
            vs = pl.ds(v * 16, 16)
            gjx[vs] = gjx[vs] - gix[vs] + ofx[vs]
            gjy[vs] = gjy[vs] - giy[vs] + ofy[vs]
            gjz[vs] = gjz[vs] - giz[vs] + ofz[vs]
            return carry2

        lax.fori_loop(0, VECS, ew, 0)

        pltpu.sync_copy(gjx, outx_hbm.at[sl])
        pltpu.sync_copy(gjy, outy_hbm.at[sl])
        pltpu.sync_copy(gjz, outz_hbm.at[sl])
        return carry

    lax.fori_loop(0, NCH, chunk, 0)


@functools.partial(
    pl.kernel,
    mesh=plsc.VectorSubcoreMesh(core_axis_name="c", subcore_axis_name="s"),
    out_type=(
        jax.ShapeDtypeStruct((E,), jnp.float32),
        jax.ShapeDtypeStruct((E,), jnp.float32),
        jax.ShapeDtypeStruct((E,), jnp.float32),
    ),
    compiler_params=pltpu.CompilerParams(needs_layout_passes=False),
    scratch_types=[
        pltpu.VMEM_SHARED((N,), jnp.float32),
        pltpu.VMEM_SHARED((N,), jnp.float32),
        pltpu.VMEM_SHARED((N,), jnp.float32),
        pltpu.VMEM((B,), jnp.int32),
        pltpu.VMEM((B,), jnp.int32),
        pltpu.VMEM((B,), jnp.float32),
        pltpu.VMEM((B,), jnp.float32),
        pltpu.VMEM((B,), jnp.float32),
        pltpu.VMEM((B,), jnp.float32),
        pltpu.VMEM((B,), jnp.float32),
        pltpu.VMEM((B,), jnp.float32),
        pltpu.VMEM((B,), jnp.float32),
        pltpu.VMEM((B,), jnp.float32),
        pltpu.VMEM((B,), jnp.float32),
        pltpu.SemaphoreType.DMA,
    ],
)
def _pairwise_sc(*refs):
    _body(*refs)


@jax.jit
def kernel(R, offsets, idx_i, idx_j):
    rx, ry, rz = R[:, 0], R[:, 1], R[:, 2]
    ox, oy, oz = offsets[:, 0], offsets[:, 1], offsets[:, 2]
    ii = idx_i.astype(jnp.int32)
    ij = idx_j.astype(jnp.int32)
    outx, outy, outz = _pairwise_sc(rx, ry, rz, ox, oy, oz, ii, ij)
    return jnp.stack([outx, outy, outz], axis=-1)
